# baseline (speedup 1.0000x reference)
import functools

import jax
import jax.numpy as jnp
from jax import lax
from jax.experimental import pallas as pl
from jax.experimental.pallas import tpu as pltpu
from jax.experimental.pallas import tpu_sc as plsc

_REG = 1e-12

_NC = 2
_NS = 16
_NW = _NC * _NS
_L = 16

_BATCH = 16384
_D = 64
_BPW = _BATCH // _NW
_CHUNK = 128
_NCHUNK = _BPW // _CHUNK
_NGROUP = _BPW // _L


def _sc_body(uids_hbm, iids_hbm, utab_hbm, itab_hbm,
             out_hbm, ssqu_hbm, ssqi_hbm,
             uidx_v, iidx_v, urows_v, irows_v, dots_v, ssq_v, sem):
    wid = lax.axis_index("s") * _NC + lax.axis_index("c")

    pltpu.sync_copy(uids_hbm.at[wid], uidx_v)
    pltpu.sync_copy(iids_hbm.at[wid], iidx_v)

    for j in range(_NCHUNK):
        pltpu.async_copy(utab_hbm.at[uidx_v.at[j]],
                         urows_v.at[pl.ds(j * _CHUNK, _CHUNK)], sem)
    for j in range(_NCHUNK):
        pltpu.async_copy(itab_hbm.at[iidx_v.at[j]],
                         irows_v.at[pl.ds(j * _CHUNK, _CHUNK)], sem)
    for j in range(2 * _NCHUNK):
        pltpu.make_async_copy(utab_hbm.at[uidx_v.at[0]],
                              urows_v.at[pl.ds(0, _CHUNK)], sem).wait()

    lanes = lax.broadcasted_iota(jnp.int32, (_L,), 0)
    zeros = jnp.zeros((_L,), jnp.float32)

    def group_body(g, carry):
        su, si = carry
        rows = g * _L + lanes
        acc = zeros
        for d in range(_D):
            cols = jnp.full((_L,), d, jnp.int32)
            u = plsc.load_gather(urows_v, [rows, cols])
            v = plsc.load_gather(irows_v, [rows, cols])
            acc = acc + u * v
            su = su + u * u
            si = si + v * v
        dots_v[pl.ds(g * _L, _L)] = acc
        return (su, si)

    su, si = lax.fori_loop(0, _NGROUP, group_body, (zeros, zeros))
    ssq_v[0, :] = su
    ssq_v[1, :] = si

    pltpu.sync_copy(dots_v, out_hbm.at[pl.ds(wid * _BPW, _BPW)])
    pltpu.sync_copy(ssq_v.at[0], ssqu_hbm.at[wid])
    pltpu.sync_copy(ssq_v.at[1], ssqi_hbm.at[wid])


@functools.partial(
    pl.kernel,
    out_type=(
        jax.ShapeDtypeStruct((_BATCH,), jnp.float32),
        jax.ShapeDtypeStruct((_NW, _L), jnp.float32),
        jax.ShapeDtypeStruct((_NW, _L), jnp.float32),
    ),
    mesh=plsc.VectorSubcoreMesh(core_axis_name="c", subcore_axis_name="s"),
    compiler_params=pltpu.CompilerParams(
        needs_layout_passes=False, use_tc_tiling_on_sc=False),
    scratch_types=(
        pltpu.VMEM((_NCHUNK, _CHUNK), jnp.int32),
        pltpu.VMEM((_NCHUNK, _CHUNK), jnp.int32),
        pltpu.VMEM((_BPW, _D), jnp.float32),
        pltpu.VMEM((_BPW, _D), jnp.float32),
        pltpu.VMEM((_BPW,), jnp.float32),
        pltpu.VMEM((2, _L), jnp.float32),
        pltpu.SemaphoreType.DMA,
    ),
)
def _funk_svd_sc(uids_hbm, iids_hbm, utab_hbm, itab_hbm,
                 out_hbm, ssqu_hbm, ssqi_hbm, *scratch):
    _sc_body(uids_hbm, iids_hbm, utab_hbm, itab_hbm,
             out_hbm, ssqu_hbm, ssqi_hbm, *scratch)


def kernel(user_ids, item_ids, user_table, item_table):
    uids = user_ids.reshape(_NW, _NCHUNK, _CHUNK)
    iids = item_ids.reshape(_NW, _NCHUNK, _CHUNK)
    dots, ssqu, ssqi = _funk_svd_sc(uids, iids, user_table, item_table)
    reg = _REG * (jnp.sqrt(jnp.sum(ssqu)) + jnp.sqrt(jnp.sum(ssqi)))
    return dots + reg


# speedup vs baseline: 2.3141x; 2.3141x over previous
import functools

import jax
import jax.numpy as jnp
from jax import lax
from jax.experimental import pallas as pl
from jax.experimental.pallas import tpu as pltpu
from jax.experimental.pallas import tpu_sc as plsc

_REG = 1e-12

_NC = 2
_NS = 16
_NW = _NC * _NS
_L = 16

_BATCH = 16384
_D = 64
_ROWS_PER_GROUP = 8
_BPW = _BATCH // _NW
_CH = 128
_NCHUNK = _BPW // _CH
_GPC = _CH // _L


def _sc_body(uids_hbm, iids_hbm, utab_hbm, itab_hbm,
             out_hbm, ssqu_hbm, ssqi_hbm,
             uid_v, iid_v, urows_v, irows_v, dots_v, ssq_v, sem):
    wid = lax.axis_index("s") * _NC + lax.axis_index("c")

    pltpu.sync_copy(uids_hbm.at[wid], uid_v)
    pltpu.sync_copy(iids_hbm.at[wid], iid_v)

    lanes = lax.broadcasted_iota(jnp.int32, (_L,), 0)
    zeros = jnp.zeros((_L,), jnp.float32)

    def chunk_body(c, carry):
        su, si = carry
        cbase = c * _CH

        def fetch_body(g, _):
            uvec = uid_v[pl.ds(cbase + g * _L, _L)]
            ivec = iid_v[pl.ds(cbase + g * _L, _L)]
            ugrp = lax.shift_right_logical(uvec, 3)
            usub = lax.bitwise_and(uvec, 7)
            igrp = lax.shift_right_logical(ivec, 3)
            isub = lax.bitwise_and(ivec, 7)
            for l in range(_L):
                row = g * _L + l
                pltpu.async_copy(utab_hbm.at[ugrp[l], usub[l]],
                                 urows_v.at[row], sem)
                pltpu.async_copy(itab_hbm.at[igrp[l], isub[l]],
                                 irows_v.at[row], sem)
            return 0

        lax.fori_loop(0, _GPC, fetch_body, 0)

        def drain_body(i, _):
            pltpu.make_async_copy(utab_hbm.at[0, 0],
                                  urows_v.at[0], sem).wait()
            return 0

        lax.fori_loop(0, 2 * _CH, drain_body, 0)

        def group_body(g, carry2):
            su, si = carry2
            rows = g * _L + lanes
            acc = zeros
            for d in range(_D):
                cols = jnp.full((_L,), d, jnp.int32)
                u = plsc.load_gather(urows_v, [rows, cols])
                v = plsc.load_gather(irows_v, [rows, cols])
                acc = acc + u * v
                su = su + u * u
                si = si + v * v
            dots_v[pl.ds(cbase + g * _L, _L)] = acc
            return (su, si)

        return lax.fori_loop(0, _GPC, group_body, (su, si))

    su, si = lax.fori_loop(0, _NCHUNK, chunk_body, (zeros, zeros))
    ssq_v[0, :] = su
    ssq_v[1, :] = si

    pltpu.sync_copy(dots_v, out_hbm.at[pl.ds(wid * _BPW, _BPW)])
    pltpu.sync_copy(ssq_v.at[0], ssqu_hbm.at[wid])
    pltpu.sync_copy(ssq_v.at[1], ssqi_hbm.at[wid])


@functools.partial(
    pl.kernel,
    out_type=(
        jax.ShapeDtypeStruct((_BATCH,), jnp.float32),
        jax.ShapeDtypeStruct((_NW, _L), jnp.float32),
        jax.ShapeDtypeStruct((_NW, _L), jnp.float32),
    ),
    mesh=plsc.VectorSubcoreMesh(core_axis_name="c", subcore_axis_name="s"),
    compiler_params=pltpu.CompilerParams(needs_layout_passes=False),
    scratch_types=(
        pltpu.VMEM((_BPW,), jnp.int32),
        pltpu.VMEM((_BPW,), jnp.int32),
        pltpu.VMEM((_CH, _D), jnp.float32),
        pltpu.VMEM((_CH, _D), jnp.float32),
        pltpu.VMEM((_BPW,), jnp.float32),
        pltpu.VMEM((2, _L), jnp.float32),
        pltpu.SemaphoreType.DMA,
    ),
)
def _funk_svd_sc(uids_hbm, iids_hbm, utab_hbm, itab_hbm,
                 out_hbm, ssqu_hbm, ssqi_hbm, *scratch):
    _sc_body(uids_hbm, iids_hbm, utab_hbm, itab_hbm,
             out_hbm, ssqu_hbm, ssqi_hbm, *scratch)


def kernel(user_ids, item_ids, user_table, item_table):
    n_groups = user_table.shape[0] // _ROWS_PER_GROUP
    utab = user_table.reshape(n_groups, _ROWS_PER_GROUP, _D)
    itab = item_table.reshape(n_groups, _ROWS_PER_GROUP, _D)
    uids = user_ids.reshape(_NW, _BPW)
    iids = item_ids.reshape(_NW, _BPW)
    dots, ssqu, ssqi = _funk_svd_sc(uids, iids, utab, itab)
    reg = _REG * (jnp.sqrt(jnp.sum(ssqu)) + jnp.sqrt(jnp.sum(ssqi)))
    return dots + reg
